# baseline (speedup 1.0000x reference)
import functools
import math

import jax
import jax.numpy as jnp
from jax import lax
from jax.experimental import pallas as pl
from jax.experimental.pallas import tpu as pltpu
from jax.experimental.pallas import tpu_sc as plsc

B, N, D = 128, 577, 768
P = N - 1
S = 16
K = 288
NSEL = K + 1
TAU = 0.5
EPS = 1e-10
SCALE = math.sqrt(N / NSEL)
BC = 8

NC, NS = 2, 16
NW = NC * NS
TOT = B * NSEL
RPW = TOT // NW
CH = 68
NCHUNK = RPW // CH


def _select_body(cls_ref, u_ref, idx_ref):
    bi = pl.program_id(0)
    logits = cls_ref[:, 1:]
    u = u_ref[...]
    g = -jnp.log(-jnp.log(u + EPS) + EPS)
    z = (logits[None, :, :] + g) / TAU
    z = z - jnp.max(z, axis=-1, keepdims=True)
    e = jnp.exp(z)
    sm = e / jnp.sum(e, axis=-1, keepdims=True)
    ps = jnp.mean(sm, axis=0)

    sb = lax.bitcast_convert_type(ps, jnp.int32)
    t = jnp.zeros((BC, 1), jnp.int32)
    for bit in range(30, -1, -1):
        cand = t | (1 << bit)
        cnt = jnp.sum((sb >= cand).astype(jnp.int32), axis=1, keepdims=True)
        t = jnp.where(cnt >= K, cand, t)

    gt = sb > t
    tie = sb == t
    m = K - jnp.sum(gt.astype(jnp.int32), axis=1, keepdims=True)

    ii = lax.broadcasted_iota(jnp.int32, (P, P), 0)
    jj = lax.broadcasted_iota(jnp.int32, (P, P), 1)
    lt = (ii <= jj).astype(jnp.float32)
    tie_rank = jnp.dot(tie.astype(jnp.float32), lt,
                       preferred_element_type=jnp.float32).astype(jnp.int32)
    sel = gt | (tie & (tie_rank <= m))
    csum = jnp.dot(sel.astype(jnp.float32), lt,
                   preferred_element_type=jnp.float32).astype(jnp.int32)

    kio = lax.broadcasted_iota(jnp.int32, (1, 1, K), 2)
    patch = jnp.sum((csum[:, :, None] <= kio).astype(jnp.int32), axis=1)

    rows = bi * BC + lax.broadcasted_iota(jnp.int32, (BC, 1), 0)
    idx_ref[...] = jnp.concatenate([rows * N, rows * N + patch + 1], axis=1)


_select = pl.pallas_call(
    _select_body,
    grid=(B // BC,),
    in_specs=[
        pl.BlockSpec((BC, N), lambda i: (i, 0)),
        pl.BlockSpec((S, BC, P), lambda i: (0, i, 0)),
    ],
    out_specs=pl.BlockSpec((BC, NSEL), lambda i: (i, 0)),
    out_shape=jax.ShapeDtypeStruct((B, NSEL), jnp.int32),
)


def _gather_body(x_hbm, idx_hbm, out_hbm, idx_v, buf0, buf1, gs0, gs1, ss0, ss1):
    wid = lax.axis_index("s") * NC + lax.axis_index("c")
    base = wid * RPW
    pltpu.sync_copy(idx_hbm.at[wid], idx_v)

    bufs = (buf0, buf1)
    gsems = (gs0, gs1)
    ssems = (ss0, ss1)
    pend_g = [None, None]
    pend_s = [None, None]

    def _scale(buf):
        def row(r, carry):
            for c16 in range(D // 16):
                sl = pl.ds(c16 * 16, 16)
                buf[r, sl] = buf[r, sl] * SCALE
            return carry
        lax.fori_loop(0, CH, row, 0)

    pend_g[0] = pltpu.async_copy(x_hbm.at[idx_v.at[0]], buf0, gs0)
    for c in range(NCHUNK):
        p = c & 1
        pend_g[p].wait()
        if c + 1 < NCHUNK:
            q = (c + 1) & 1
            if pend_s[q] is not None:
                pend_s[q].wait()
            pend_g[q] = pltpu.async_copy(x_hbm.at[idx_v.at[c + 1]], bufs[q], gsems[q])
        _scale(bufs[p])
        pend_s[p] = pltpu.async_copy(
            bufs[p], out_hbm.at[pl.ds(base + c * CH, CH)], ssems[p])
    pend_s[0].wait()
    pend_s[1].wait()


@functools.lru_cache(maxsize=1)
def _make_gather():
    return functools.partial(
        pl.kernel,
        mesh=plsc.VectorSubcoreMesh(core_axis_name="c", subcore_axis_name="s"),
        compiler_params=pltpu.CompilerParams(use_tc_tiling_on_sc=False),
        out_type=jax.ShapeDtypeStruct((TOT, D), jnp.float32),
        scratch_types=[
            pltpu.VMEM((NCHUNK, CH), jnp.int32),
            pltpu.VMEM((CH, D), jnp.float32),
            pltpu.VMEM((CH, D), jnp.float32),
            pltpu.SemaphoreType.DMA,
            pltpu.SemaphoreType.DMA,
            pltpu.SemaphoreType.DMA,
            pltpu.SemaphoreType.DMA,
        ],
    )(_gather_body)


def kernel(x, cls_attn, u):
    flat_idx = _select(cls_attn, u)
    idx3 = flat_idx.reshape(NW, NCHUNK, CH)
    x2 = x.reshape(B * N, D)
    out = _make_gather()(x2, idx3)
    return out.reshape(B, NSEL, D)
